# baseline (speedup 1.0000x reference)
import math

import jax
import jax.numpy as jnp
from jax import lax
from jax.experimental import pallas as pl
from jax.experimental.pallas import tpu as pltpu
from jax.experimental.pallas import tpu_sc as plsc

D_MODEL = 128
SEQ_L = 200
CHUNK = 80
POS_BUF = SEQ_L + CHUNK - 40
LANES = 16
NUM_CORES = 2
NUM_SUBCORES = 16
NUM_WORKERS = NUM_CORES * NUM_SUBCORES
NBUF = 5
SCALE_BLK = 10000
SCALE = math.sqrt(D_MODEL)


def _scale_body(w_ref, o_ref):
    o_ref[...] = w_ref[...] * math.sqrt(D_MODEL)


def _sc_body(x2_hbm, tab_hbm, pos_hbm, out_hbm, *scratch):
    idx_all, pos_v = scratch[0], scratch[1]
    rbufs = scratch[2 : 2 + NBUF]
    gsems = scratch[2 + NBUF : 2 + 2 * NBUF]
    ssems = scratch[2 + 2 * NBUF : 2 + 3 * NBUF]

    n_chunks = x2_hbm.shape[0] // NUM_WORKERS
    wid = lax.axis_index("s") * NUM_CORES + lax.axis_index("c")
    cbase = wid * n_chunks

    pltpu.sync_copy(pos_hbm.at[pl.ds(0, SEQ_L)], pos_v.at[pl.ds(0, SEQ_L)])
    pltpu.sync_copy(
        pos_hbm.at[pl.ds(0, POS_BUF - SEQ_L)], pos_v.at[pl.ds(SEQ_L, POS_BUF - SEQ_L)]
    )
    pltpu.sync_copy(x2_hbm.at[pl.ds(cbase, n_chunks)], idx_all)

    pltpu.async_copy(tab_hbm.at[idx_all.at[0]], rbufs[0], gsems[0])
    pltpu.async_copy(tab_hbm.at[idx_all.at[1]], rbufs[1], gsems[1])
    pltpu.async_copy(tab_hbm.at[idx_all.at[2]], rbufs[2], gsems[2])

    def outer(o, carry):
        for j in range(NBUF):
            t = NBUF * o + j
            p = j
            q = (j + 3) % NBUF

            @pl.when(t + 3 < n_chunks)
            def _prefetch():
                @pl.when(t >= 2)
                def _drain_store():
                    pltpu.make_async_copy(
                        rbufs[q], out_hbm.at[pl.ds(0, CHUNK)], ssems[q]
                    ).wait()

                pltpu.async_copy(tab_hbm.at[idx_all.at[t + 3]], rbufs[q], gsems[q])

            pltpu.make_async_copy(
                tab_hbm.at[pl.ds(0, CHUNK)], rbufs[p], gsems[p]
            ).wait()

            phase = (j * CHUNK) % SEQ_L
            rbuf = rbufs[p]

            @plsc.parallel_loop(0, CHUNK, step=1, unroll=10)
            def row_body(r):
                for c in range(D_MODEL // LANES):
                    sl = pl.ds(c * LANES, LANES)
                    rbuf[r, sl] = rbuf[r, sl] * SCALE + pos_v[phase + r, sl]

            pltpu.async_copy(
                rbufs[p], out_hbm.at[pl.ds((cbase + t) * CHUNK, CHUNK)], ssems[p]
            )
        return carry

    lax.fori_loop(0, n_chunks // NBUF, outer, 0)

    for j in range(NBUF):
        pltpu.make_async_copy(
            rbufs[j], out_hbm.at[pl.ds(0, CHUNK)], ssems[j]
        ).wait()


def kernel(x, emb_weight, pos_encoding):
    b, l = x.shape
    v, d = emb_weight.shape
    x2 = x.reshape(b * l // CHUNK, CHUNK)

    mesh = plsc.VectorSubcoreMesh(
        core_axis_name="c",
        subcore_axis_name="s",
        num_cores=NUM_CORES,
        num_subcores=NUM_SUBCORES,
    )
    n_chunks = x2.shape[0] // NUM_WORKERS
    run = pl.kernel(
        _sc_body,
        out_type=jax.ShapeDtypeStruct((b * l, d), jnp.float32),
        mesh=mesh,
        scratch_types=(
            [
                pltpu.VMEM((n_chunks, CHUNK), jnp.int32),
                pltpu.VMEM((POS_BUF, d), jnp.float32),
            ]
            + [pltpu.VMEM((CHUNK, d), jnp.float32) for _ in range(NBUF)]
            + [pltpu.SemaphoreType.DMA for _ in range(2 * NBUF)]
        ),
    )
    out = run(x2, emb_weight, pos_encoding)
    return out.reshape(b, l, d)


# speedup vs baseline: 1.1220x; 1.1220x over previous
import math

import jax
import jax.numpy as jnp
from jax import lax
from jax.experimental import pallas as pl
from jax.experimental.pallas import tpu as pltpu
from jax.experimental.pallas import tpu_sc as plsc

D_MODEL = 128
SEQ_L = 200
CHUNK = 80
POS_BUF = SEQ_L + CHUNK - 40
LANES = 16
NUM_CORES = 2
NUM_SUBCORES = 16
NUM_WORKERS = NUM_CORES * NUM_SUBCORES
NBUF = 5
SCALE_BLK = 10000
SCALE = math.sqrt(D_MODEL)


def _scale_body(w_ref, o_ref):
    o_ref[...] = w_ref[...] * math.sqrt(D_MODEL)


def _sc_body(x2_hbm, tab_hbm, pos_hbm, out_hbm, *scratch):
    idx_all, pos_v = scratch[0], scratch[1]
    rbufs = scratch[2 : 2 + NBUF]
    gsems = scratch[2 + NBUF : 2 + 2 * NBUF]
    ssems = scratch[2 + 2 * NBUF : 2 + 3 * NBUF]

    n_chunks = x2_hbm.shape[0] // NUM_WORKERS
    wid = lax.axis_index("s") * NUM_CORES + lax.axis_index("c")
    cbase = wid * n_chunks

    pltpu.sync_copy(pos_hbm.at[pl.ds(0, SEQ_L)], pos_v.at[pl.ds(0, SEQ_L)])
    pltpu.sync_copy(
        pos_hbm.at[pl.ds(0, POS_BUF - SEQ_L)], pos_v.at[pl.ds(SEQ_L, POS_BUF - SEQ_L)]
    )
    pltpu.sync_copy(x2_hbm.at[pl.ds(cbase, n_chunks)], idx_all)

    pltpu.async_copy(tab_hbm.at[idx_all.at[0]], rbufs[0], gsems[0])
    pltpu.async_copy(tab_hbm.at[idx_all.at[1]], rbufs[1], gsems[1])
    pltpu.async_copy(tab_hbm.at[idx_all.at[2]], rbufs[2], gsems[2])

    def outer(o, carry):
        for j in range(NBUF):
            t = NBUF * o + j
            p = j
            q = (j + 3) % NBUF

            @pl.when(t + 3 < n_chunks)
            def _prefetch():
                @pl.when(t >= 2)
                def _drain_store():
                    pltpu.make_async_copy(
                        rbufs[q], out_hbm.at[pl.ds(0, CHUNK)], ssems[q]
                    ).wait()

                pltpu.async_copy(tab_hbm.at[idx_all.at[t + 3]], rbufs[q], gsems[q])

            pltpu.make_async_copy(
                tab_hbm.at[pl.ds(0, CHUNK)], rbufs[p], gsems[p]
            ).wait()

            phase = (j * CHUNK) % SEQ_L
            rbuf = rbufs[p]
            half = CHUNK // 2

            @plsc.parallel_loop(0, half, step=1, unroll=8)
            def row_body_a(r):
                for c in range(D_MODEL // LANES):
                    sl = pl.ds(c * LANES, LANES)
                    rbuf[r, sl] = rbuf[r, sl] * SCALE + pos_v[phase + r, sl]

            pltpu.async_copy(
                rbuf.at[pl.ds(0, half)],
                out_hbm.at[pl.ds((cbase + t) * CHUNK, half)],
                ssems[p],
            )

            @plsc.parallel_loop(half, CHUNK, step=1, unroll=8)
            def row_body_b(r):
                for c in range(D_MODEL // LANES):
                    sl = pl.ds(c * LANES, LANES)
                    rbuf[r, sl] = rbuf[r, sl] * SCALE + pos_v[phase + r, sl]

            pltpu.async_copy(
                rbuf.at[pl.ds(half, half)],
                out_hbm.at[pl.ds((cbase + t) * CHUNK + half, half)],
                ssems[p],
            )
        return carry

    lax.fori_loop(0, n_chunks // NBUF, outer, 0)

    for j in range(NBUF):
        pltpu.make_async_copy(
            rbufs[j], out_hbm.at[pl.ds(0, CHUNK)], ssems[j]
        ).wait()


def kernel(x, emb_weight, pos_encoding):
    b, l = x.shape
    v, d = emb_weight.shape
    x2 = x.reshape(b * l // CHUNK, CHUNK)

    mesh = plsc.VectorSubcoreMesh(
        core_axis_name="c",
        subcore_axis_name="s",
        num_cores=NUM_CORES,
        num_subcores=NUM_SUBCORES,
    )
    n_chunks = x2.shape[0] // NUM_WORKERS
    run = pl.kernel(
        _sc_body,
        out_type=jax.ShapeDtypeStruct((b * l, d), jnp.float32),
        mesh=mesh,
        scratch_types=(
            [
                pltpu.VMEM((n_chunks, CHUNK), jnp.int32),
                pltpu.VMEM((POS_BUF, d), jnp.float32),
            ]
            + [pltpu.VMEM((CHUNK, d), jnp.float32) for _ in range(NBUF)]
            + [pltpu.SemaphoreType.DMA for _ in range(2 * NBUF)]
        ),
    )
    out = run(x2, emb_weight, pos_encoding)
    return out.reshape(b, l, d)


# speedup vs baseline: 1.3099x; 1.1675x over previous
import math

import jax
import jax.numpy as jnp
from jax import lax
from jax.experimental import pallas as pl
from jax.experimental.pallas import tpu as pltpu
from jax.experimental.pallas import tpu_sc as plsc

D_MODEL = 128
SEQ_L = 200
CHUNK = 80
POS_BUF = SEQ_L + CHUNK - 40
LANES = 16
NUM_CORES = 2
NUM_SUBCORES = 16
NUM_WORKERS = NUM_CORES * NUM_SUBCORES
NBUF = 8
LOOKAHEAD = 5
PHASE_PERIOD = SEQ_L // math.gcd(CHUNK, SEQ_L)
SCALE = math.sqrt(D_MODEL)


def _sc_body(x2_hbm, tab_hbm, pos_hbm, out_hbm, *scratch):
    pos_v = scratch[0]
    rbufs = scratch[1 : 1 + NBUF]
    ibufs = scratch[1 + NBUF : 1 + 2 * NBUF]
    gsems = scratch[1 + 2 * NBUF : 1 + 3 * NBUF]
    ssems = scratch[1 + 3 * NBUF : 1 + 4 * NBUF]
    isems = scratch[1 + 4 * NBUF : 1 + 5 * NBUF]

    n_chunks = x2_hbm.shape[0] // NUM_WORKERS
    wid = lax.axis_index("s") * NUM_CORES + lax.axis_index("c")
    cbase = wid * n_chunks

    pltpu.sync_copy(pos_hbm.at[pl.ds(0, SEQ_L)], pos_v.at[pl.ds(0, SEQ_L)])
    pltpu.sync_copy(
        pos_hbm.at[pl.ds(0, POS_BUF - SEQ_L)], pos_v.at[pl.ds(SEQ_L, POS_BUF - SEQ_L)]
    )

    for i in range(NBUF):
        pltpu.async_copy(x2_hbm.at[pl.ds(cbase + i, 1)], ibufs[i], isems[i])
    for i in range(LOOKAHEAD):
        pltpu.make_async_copy(x2_hbm.at[pl.ds(0, 1)], ibufs[i], isems[i]).wait()
        pltpu.async_copy(tab_hbm.at[ibufs[i].at[0]], rbufs[i], gsems[i])

    def outer(o, carry):
        for j in range(NBUF):
            t = NBUF * o + j
            p = j
            q = (j + LOOKAHEAD) % NBUF

            @pl.when(t + LOOKAHEAD < n_chunks)
            def _prefetch():
                @pl.when(t >= NBUF - LOOKAHEAD)
                def _drain_store():
                    pltpu.make_async_copy(
                        rbufs[q], out_hbm.at[pl.ds(0, CHUNK)], ssems[q]
                    ).wait()

                pltpu.make_async_copy(
                    x2_hbm.at[pl.ds(0, 1)], ibufs[q], isems[q]
                ).wait()

                pltpu.async_copy(tab_hbm.at[ibufs[q].at[0]], rbufs[q], gsems[q])

            pltpu.make_async_copy(
                tab_hbm.at[pl.ds(0, CHUNK)], rbufs[p], gsems[p]
            ).wait()

            @pl.when(t + NBUF < n_chunks)
            def _idx_prefetch():
                pltpu.async_copy(
                    x2_hbm.at[pl.ds(cbase + t + NBUF, 1)], ibufs[p], isems[p]
                )

            phase = lax.rem(lax.rem(t, PHASE_PERIOD) * CHUNK, SEQ_L)
            rbuf = rbufs[p]

            @plsc.parallel_loop(0, CHUNK, step=1, unroll=8)
            def row_body(r):
                for c in range(D_MODEL // LANES):
                    sl = pl.ds(c * LANES, LANES)
                    rbuf[r, sl] = rbuf[r, sl] * SCALE + pos_v[phase + r, sl]

            pltpu.async_copy(
                rbufs[p], out_hbm.at[pl.ds((cbase + t) * CHUNK, CHUNK)], ssems[p]
            )
        return carry

    lax.fori_loop(0, n_chunks // NBUF, outer, 0)

    for j in range(NBUF):
        pltpu.make_async_copy(
            rbufs[j], out_hbm.at[pl.ds(0, CHUNK)], ssems[j]
        ).wait()


def kernel(x, emb_weight, pos_encoding):
    b, l = x.shape
    v, d = emb_weight.shape
    x2 = x.reshape(b * l // CHUNK, CHUNK)

    mesh = plsc.VectorSubcoreMesh(
        core_axis_name="c",
        subcore_axis_name="s",
        num_cores=NUM_CORES,
        num_subcores=NUM_SUBCORES,
    )
    run = pl.kernel(
        _sc_body,
        out_type=jax.ShapeDtypeStruct((b * l, d), jnp.float32),
        mesh=mesh,
        scratch_types=(
            [pltpu.VMEM((POS_BUF, d), jnp.float32)]
            + [pltpu.VMEM((CHUNK, d), jnp.float32) for _ in range(NBUF)]
            + [pltpu.VMEM((1, CHUNK), jnp.int32) for _ in range(NBUF)]
            + [pltpu.SemaphoreType.DMA for _ in range(3 * NBUF)]
        ),
    )
    out = run(x2, emb_weight, pos_encoding)
    return out.reshape(b, l, d)


# speedup vs baseline: 1.3111x; 1.0009x over previous
import math

import jax
import jax.numpy as jnp
from jax import lax
from jax.experimental import pallas as pl
from jax.experimental.pallas import tpu as pltpu
from jax.experimental.pallas import tpu_sc as plsc

D_MODEL = 128
SEQ_L = 200
CHUNK = 128
POS_BUF = SEQ_L + CHUNK - 8
LANES = 16
NUM_CORES = 2
NUM_SUBCORES = 16
NUM_WORKERS = NUM_CORES * NUM_SUBCORES
NBUF = 5
LOOKAHEAD = 3
PHASE_PERIOD = SEQ_L // math.gcd(CHUNK, SEQ_L)
SCALE = math.sqrt(D_MODEL)


def _sc_body(x2_hbm, tab_hbm, pos_hbm, out_hbm, *scratch):
    pos_v = scratch[0]
    rbufs = scratch[1 : 1 + NBUF]
    ibufs = scratch[1 + NBUF : 1 + 2 * NBUF]
    gsems = scratch[1 + 2 * NBUF : 1 + 3 * NBUF]
    ssems = scratch[1 + 3 * NBUF : 1 + 4 * NBUF]
    isems = scratch[1 + 4 * NBUF : 1 + 5 * NBUF]

    n_chunks = x2_hbm.shape[0] // NUM_WORKERS
    wid = lax.axis_index("s") * NUM_CORES + lax.axis_index("c")
    cbase = wid * n_chunks

    pltpu.sync_copy(pos_hbm.at[pl.ds(0, SEQ_L)], pos_v.at[pl.ds(0, SEQ_L)])
    pltpu.sync_copy(
        pos_hbm.at[pl.ds(0, POS_BUF - SEQ_L)], pos_v.at[pl.ds(SEQ_L, POS_BUF - SEQ_L)]
    )

    for i in range(NBUF):
        pltpu.async_copy(x2_hbm.at[pl.ds(cbase + i, 1)], ibufs[i], isems[i])
    for i in range(LOOKAHEAD):
        pltpu.make_async_copy(x2_hbm.at[pl.ds(0, 1)], ibufs[i], isems[i]).wait()
        pltpu.async_copy(tab_hbm.at[ibufs[i].at[0]], rbufs[i], gsems[i])

    def outer(o, carry):
        for j in range(NBUF):
            t = NBUF * o + j
            p = j
            q = (j + LOOKAHEAD) % NBUF

            @pl.when(t + LOOKAHEAD < n_chunks)
            def _prefetch():
                @pl.when(t >= NBUF - LOOKAHEAD)
                def _drain_store():
                    pltpu.make_async_copy(
                        rbufs[q], out_hbm.at[pl.ds(0, CHUNK)], ssems[q]
                    ).wait()

                pltpu.make_async_copy(
                    x2_hbm.at[pl.ds(0, 1)], ibufs[q], isems[q]
                ).wait()

                pltpu.async_copy(tab_hbm.at[ibufs[q].at[0]], rbufs[q], gsems[q])

            pltpu.make_async_copy(
                tab_hbm.at[pl.ds(0, CHUNK)], rbufs[p], gsems[p]
            ).wait()

            @pl.when(t + NBUF < n_chunks)
            def _idx_prefetch():
                pltpu.async_copy(
                    x2_hbm.at[pl.ds(cbase + t + NBUF, 1)], ibufs[p], isems[p]
                )

            phase = lax.rem(lax.rem(t, PHASE_PERIOD) * CHUNK, SEQ_L)
            rbuf = rbufs[p]

            @plsc.parallel_loop(0, CHUNK, step=1, unroll=8)
            def row_body(r):
                for c in range(D_MODEL // LANES):
                    sl = pl.ds(c * LANES, LANES)
                    rbuf[r, sl] = rbuf[r, sl] * SCALE + pos_v[phase + r, sl]

            pltpu.async_copy(
                rbufs[p], out_hbm.at[pl.ds((cbase + t) * CHUNK, CHUNK)], ssems[p]
            )
        return carry

    lax.fori_loop(0, n_chunks // NBUF, outer, 0)

    for j in range(NBUF):
        pltpu.make_async_copy(
            rbufs[j], out_hbm.at[pl.ds(0, CHUNK)], ssems[j]
        ).wait()


def kernel(x, emb_weight, pos_encoding):
    b, l = x.shape
    v, d = emb_weight.shape
    x2 = x.reshape(b * l // CHUNK, CHUNK)

    mesh = plsc.VectorSubcoreMesh(
        core_axis_name="c",
        subcore_axis_name="s",
        num_cores=NUM_CORES,
        num_subcores=NUM_SUBCORES,
    )
    run = pl.kernel(
        _sc_body,
        out_type=jax.ShapeDtypeStruct((b * l, d), jnp.float32),
        mesh=mesh,
        scratch_types=(
            [pltpu.VMEM((POS_BUF, d), jnp.float32)]
            + [pltpu.VMEM((CHUNK, d), jnp.float32) for _ in range(NBUF)]
            + [pltpu.VMEM((1, CHUNK), jnp.int32) for _ in range(NBUF)]
            + [pltpu.SemaphoreType.DMA for _ in range(3 * NBUF)]
        ),
    )
    out = run(x2, emb_weight, pos_encoding)
    return out.reshape(b, l, d)


# speedup vs baseline: 1.3155x; 1.0034x over previous
import math

import jax
import jax.numpy as jnp
from jax import lax
from jax.experimental import pallas as pl
from jax.experimental.pallas import tpu as pltpu
from jax.experimental.pallas import tpu_sc as plsc

D_MODEL = 128
SEQ_L = 200
CHUNK = 128
POS_BUF = SEQ_L + CHUNK - 8
LANES = 16
NUM_CORES = 2
NUM_SUBCORES = 16
NUM_WORKERS = NUM_CORES * NUM_SUBCORES
NBUF = 5
LOOKAHEAD = 3
PHASE_PERIOD = SEQ_L // math.gcd(CHUNK, SEQ_L)
SCALE = math.sqrt(D_MODEL)


def _sc_body(x2_hbm, tab_hbm, pos_hbm, out_hbm, *scratch):
    pos_v = scratch[0]
    rbufs = scratch[1 : 1 + NBUF]
    ibufs = scratch[1 + NBUF : 1 + 2 * NBUF]
    gsems = scratch[1 + 2 * NBUF : 1 + 3 * NBUF]
    ssems = scratch[1 + 3 * NBUF : 1 + 4 * NBUF]
    isems = scratch[1 + 4 * NBUF : 1 + 5 * NBUF]
    psems = scratch[1 + 5 * NBUF : 3 + 5 * NBUF]

    n_chunks = x2_hbm.shape[0] // NUM_WORKERS
    wid = lax.axis_index("s") * NUM_CORES + lax.axis_index("c")
    cbase = wid * n_chunks

    pltpu.async_copy(pos_hbm.at[pl.ds(0, SEQ_L)], pos_v.at[pl.ds(0, SEQ_L)], psems[0])
    pltpu.async_copy(
        pos_hbm.at[pl.ds(0, POS_BUF - SEQ_L)],
        pos_v.at[pl.ds(SEQ_L, POS_BUF - SEQ_L)],
        psems[1],
    )

    for i in range(NBUF):
        pltpu.async_copy(x2_hbm.at[pl.ds(cbase + i, 1)], ibufs[i], isems[i])
    for i in range(LOOKAHEAD):
        pltpu.make_async_copy(x2_hbm.at[pl.ds(0, 1)], ibufs[i], isems[i]).wait()
        pltpu.async_copy(tab_hbm.at[ibufs[i].at[0]], rbufs[i], gsems[i])
    pltpu.make_async_copy(
        pos_hbm.at[pl.ds(0, SEQ_L)], pos_v.at[pl.ds(0, SEQ_L)], psems[0]
    ).wait()
    pltpu.make_async_copy(
        pos_hbm.at[pl.ds(0, POS_BUF - SEQ_L)],
        pos_v.at[pl.ds(SEQ_L, POS_BUF - SEQ_L)],
        psems[1],
    ).wait()

    def outer(o, carry):
        for j in range(NBUF):
            t = NBUF * o + j
            p = j
            q = (j + LOOKAHEAD) % NBUF

            @pl.when(t + LOOKAHEAD < n_chunks)
            def _prefetch():
                @pl.when(t >= NBUF - LOOKAHEAD)
                def _drain_store():
                    pltpu.make_async_copy(
                        rbufs[q], out_hbm.at[pl.ds(0, CHUNK)], ssems[q]
                    ).wait()

                pltpu.make_async_copy(
                    x2_hbm.at[pl.ds(0, 1)], ibufs[q], isems[q]
                ).wait()

                pltpu.async_copy(tab_hbm.at[ibufs[q].at[0]], rbufs[q], gsems[q])

            pltpu.make_async_copy(
                tab_hbm.at[pl.ds(0, CHUNK)], rbufs[p], gsems[p]
            ).wait()

            @pl.when(t + NBUF < n_chunks)
            def _idx_prefetch():
                pltpu.async_copy(
                    x2_hbm.at[pl.ds(cbase + t + NBUF, 1)], ibufs[p], isems[p]
                )

            phase = lax.rem(lax.rem(t, PHASE_PERIOD) * CHUNK, SEQ_L)
            rbuf = rbufs[p]

            @plsc.parallel_loop(0, CHUNK, step=1, unroll=8)
            def row_body(r):
                for c in range(D_MODEL // LANES):
                    sl = pl.ds(c * LANES, LANES)
                    rbuf[r, sl] = rbuf[r, sl] * SCALE + pos_v[phase + r, sl]

            pltpu.async_copy(
                rbufs[p], out_hbm.at[pl.ds((cbase + t) * CHUNK, CHUNK)], ssems[p]
            )
        return carry

    lax.fori_loop(0, n_chunks // NBUF, outer, 0)

    for j in range(NBUF):
        pltpu.make_async_copy(
            rbufs[j], out_hbm.at[pl.ds(0, CHUNK)], ssems[j]
        ).wait()


def kernel(x, emb_weight, pos_encoding):
    b, l = x.shape
    v, d = emb_weight.shape
    x2 = x.reshape(b * l // CHUNK, CHUNK)

    mesh = plsc.VectorSubcoreMesh(
        core_axis_name="c",
        subcore_axis_name="s",
        num_cores=NUM_CORES,
        num_subcores=NUM_SUBCORES,
    )
    run = pl.kernel(
        _sc_body,
        out_type=jax.ShapeDtypeStruct((b * l, d), jnp.float32),
        mesh=mesh,
        scratch_types=(
            [pltpu.VMEM((POS_BUF, d), jnp.float32)]
            + [pltpu.VMEM((CHUNK, d), jnp.float32) for _ in range(NBUF)]
            + [pltpu.VMEM((1, CHUNK), jnp.int32) for _ in range(NBUF)]
            + [pltpu.SemaphoreType.DMA for _ in range(3 * NBUF + 2)]
        ),
    )
    out = run(x2, emb_weight, pos_encoding)
    return out.reshape(b, l, d)


# speedup vs baseline: 1.3203x; 1.0037x over previous
import math

import jax
import jax.numpy as jnp
from jax import lax
from jax.experimental import pallas as pl
from jax.experimental.pallas import tpu as pltpu
from jax.experimental.pallas import tpu_sc as plsc

D_MODEL = 128
SEQ_L = 200
CHUNK = 128
POS_BUF = SEQ_L + CHUNK - 8
LANES = 16
NUM_CORES = 2
NUM_SUBCORES = 16
NUM_WORKERS = NUM_CORES * NUM_SUBCORES
NBUF = 5
LOOKAHEAD = 2
PHASE_PERIOD = SEQ_L // math.gcd(CHUNK, SEQ_L)
SCALE = math.sqrt(D_MODEL)


def _sc_body(x2_hbm, tab_hbm, pos_hbm, out_hbm, *scratch):
    pos_v = scratch[0]
    rbufs = scratch[1 : 1 + NBUF]
    ibufs = scratch[1 + NBUF : 1 + 2 * NBUF]
    gsems = scratch[1 + 2 * NBUF : 1 + 3 * NBUF]
    ssems = scratch[1 + 3 * NBUF : 1 + 4 * NBUF]
    isems = scratch[1 + 4 * NBUF : 1 + 5 * NBUF]
    psems = scratch[1 + 5 * NBUF : 3 + 5 * NBUF]

    n_chunks = x2_hbm.shape[0] // NUM_WORKERS
    wid = lax.axis_index("s") * NUM_CORES + lax.axis_index("c")
    cbase = wid * n_chunks

    pltpu.async_copy(pos_hbm.at[pl.ds(0, SEQ_L)], pos_v.at[pl.ds(0, SEQ_L)], psems[0])
    pltpu.async_copy(
        pos_hbm.at[pl.ds(0, POS_BUF - SEQ_L)],
        pos_v.at[pl.ds(SEQ_L, POS_BUF - SEQ_L)],
        psems[1],
    )

    for i in range(NBUF):
        pltpu.async_copy(x2_hbm.at[pl.ds(cbase + i, 1)], ibufs[i], isems[i])
    for i in range(LOOKAHEAD):
        pltpu.make_async_copy(x2_hbm.at[pl.ds(0, 1)], ibufs[i], isems[i]).wait()
        pltpu.async_copy(tab_hbm.at[ibufs[i].at[0]], rbufs[i], gsems[i])
    pltpu.make_async_copy(
        pos_hbm.at[pl.ds(0, SEQ_L)], pos_v.at[pl.ds(0, SEQ_L)], psems[0]
    ).wait()
    pltpu.make_async_copy(
        pos_hbm.at[pl.ds(0, POS_BUF - SEQ_L)],
        pos_v.at[pl.ds(SEQ_L, POS_BUF - SEQ_L)],
        psems[1],
    ).wait()

    def outer(o, carry):
        for j in range(NBUF):
            t = NBUF * o + j
            p = j
            q = (j + LOOKAHEAD) % NBUF

            @pl.when(t + LOOKAHEAD < n_chunks)
            def _prefetch():
                @pl.when(t >= NBUF - LOOKAHEAD)
                def _drain_store():
                    pltpu.make_async_copy(
                        rbufs[q], out_hbm.at[pl.ds(0, CHUNK)], ssems[q]
                    ).wait()

                pltpu.make_async_copy(
                    x2_hbm.at[pl.ds(0, 1)], ibufs[q], isems[q]
                ).wait()

                pltpu.async_copy(tab_hbm.at[ibufs[q].at[0]], rbufs[q], gsems[q])

            pltpu.make_async_copy(
                tab_hbm.at[pl.ds(0, CHUNK)], rbufs[p], gsems[p]
            ).wait()

            @pl.when(t + NBUF < n_chunks)
            def _idx_prefetch():
                pltpu.async_copy(
                    x2_hbm.at[pl.ds(cbase + t + NBUF, 1)], ibufs[p], isems[p]
                )

            phase = lax.rem(lax.rem(t, PHASE_PERIOD) * CHUNK, SEQ_L)
            rbuf = rbufs[p]

            @plsc.parallel_loop(0, CHUNK, step=1, unroll=8)
            def row_body(r):
                for c in range(D_MODEL // LANES):
                    sl = pl.ds(c * LANES, LANES)
                    rbuf[r, sl] = rbuf[r, sl] * SCALE + pos_v[phase + r, sl]

            pltpu.async_copy(
                rbufs[p], out_hbm.at[pl.ds((cbase + t) * CHUNK, CHUNK)], ssems[p]
            )
        return carry

    lax.fori_loop(0, n_chunks // NBUF, outer, 0)

    for j in range(NBUF):
        pltpu.make_async_copy(
            rbufs[j], out_hbm.at[pl.ds(0, CHUNK)], ssems[j]
        ).wait()


def kernel(x, emb_weight, pos_encoding):
    b, l = x.shape
    v, d = emb_weight.shape
    x2 = x.reshape(b * l // CHUNK, CHUNK)

    mesh = plsc.VectorSubcoreMesh(
        core_axis_name="c",
        subcore_axis_name="s",
        num_cores=NUM_CORES,
        num_subcores=NUM_SUBCORES,
    )
    run = pl.kernel(
        _sc_body,
        out_type=jax.ShapeDtypeStruct((b * l, d), jnp.float32),
        mesh=mesh,
        scratch_types=(
            [pltpu.VMEM((POS_BUF, d), jnp.float32)]
            + [pltpu.VMEM((CHUNK, d), jnp.float32) for _ in range(NBUF)]
            + [pltpu.VMEM((1, CHUNK), jnp.int32) for _ in range(NBUF)]
            + [pltpu.SemaphoreType.DMA for _ in range(3 * NBUF + 2)]
        ),
    )
    out = run(x2, emb_weight, pos_encoding)
    return out.reshape(b, l, d)
